# baseline (speedup 1.0000x reference)
import jax
import jax.numpy as jnp
from jax import lax
from jax.experimental import pallas as pl
from jax.experimental.pallas import tpu as pltpu
from jax.experimental.pallas import tpu_sc as plsc

N_NODES = 10000
N_EDGES = 320000
D_IN = 128
D_OUT = 2
N_SEL = 9
BN_EPS = 1e-5

NC = 2
NS = 16
LANES = 16
NW = NC * NS
E_PER_W = N_EDGES // NW
CHUNK = 2000
N_CHUNK = E_PER_W // CHUNK
UNROLL = 16
N_PAD = 10240
NROW = N_PAD // 128
TAB_LEN = N_SEL * N_PAD
NB = 2048
GRID1 = N_PAD // NB


def _pack_kernel(x_ref, w_ref, o_ref):
    dn = (((1,), (1,)), ((), ()))
    a = lax.dot_general(w_ref[...], x_ref[...], dn,
                        preferred_element_type=jnp.float32)
    u0 = lax.bitcast_convert_type(a[:N_SEL].astype(jnp.bfloat16),
                                  jnp.uint16).astype(jnp.uint32)
    u1 = lax.bitcast_convert_type(a[N_SEL:].astype(jnp.bfloat16),
                                  jnp.uint16).astype(jnp.uint32)
    packed = lax.bitcast_convert_type((u1 << 16) | u0, jnp.int32)
    o_ref[...] = packed.reshape(N_SEL, NB // 128, 128)


def _build_table(x, W):
    w01 = W.transpose(1, 0, 2).reshape(D_OUT * N_SEL, D_IN)
    return pl.pallas_call(
        _pack_kernel,
        grid=(GRID1,),
        in_specs=[
            pl.BlockSpec((NB, D_IN), lambda i: (i, 0)),
            pl.BlockSpec((D_OUT * N_SEL, D_IN), lambda i: (0, 0)),
        ],
        out_specs=pl.BlockSpec((N_SEL, NB // 128, 128), lambda i: (0, i, 0)),
        out_shape=jax.ShapeDtypeStruct((N_SEL, NROW, 128), jnp.int32),
    )(x, w01)


def _edge_kernel(table_hbm, ei_hbm, sel_hbm, part_hbm,
                 table_v, acc0, acc1,
                 src_a, dst_a, sel_a, src_c, dst_c, sel_c,
                 sem_a, sem_c, sem_out):
    wid = lax.axis_index("s") * NC + lax.axis_index("c")
    bufs = ((src_a, dst_a, sel_a), (src_c, dst_c, sel_c))
    sems = (sem_a, sem_c)

    def issue(c, k):
        base = wid * E_PER_W + c * CHUNK
        return (
            pltpu.async_copy(ei_hbm.at[0, pl.ds(base, CHUNK)], bufs[k][0], sems[k]),
            pltpu.async_copy(ei_hbm.at[1, pl.ds(base, CHUNK)], bufs[k][1], sems[k]),
            pltpu.async_copy(sel_hbm.at[pl.ds(base, CHUNK)], bufs[k][2], sems[k]),
        )

    pend = issue(0, 0)
    tab_h = pltpu.async_copy(table_hbm, table_v, sem_out)

    zeros = jnp.zeros((LANES,), jnp.float32)

    @plsc.parallel_loop(0, N_PAD // LANES, step=1, unroll=8)
    def zero_body(i):
        acc0[pl.ds(i * LANES, LANES)] = zeros
        acc1[pl.ds(i * LANES, LANES)] = zeros

    tab_h.wait()

    for c in range(N_CHUNK):
        k = c & 1
        for h in pend:
            h.wait()
        if c + 1 < N_CHUNK:
            pend = issue(c + 1, 1 - k)
        src_b, dst_b, sel_b = bufs[k]

        @plsc.parallel_loop(0, CHUNK // LANES, step=1, unroll=UNROLL)
        def body(i):
            o = i * LANES
            s = src_b[pl.ds(o, LANES)]
            d = dst_b[pl.ds(o, LANES)]
            q = sel_b[pl.ds(o, LANES)]
            g = q * N_PAD + s
            w = plsc.load_gather(table_v, [g])
            c0 = plsc.bitcast(w << 16, jnp.float32)
            c1 = plsc.bitcast(w & jnp.int32(-65536), jnp.float32)
            plsc.addupdate_scatter(acc0, [d], c0)
            plsc.addupdate_scatter(acc1, [d], c1)

    o1 = pltpu.async_copy(acc0, part_hbm.at[wid, 0], sem_out)
    o2 = pltpu.async_copy(acc1, part_hbm.at[wid, 1], sem_out)
    o1.wait()
    o2.wait()


def _scatter_edges(table_flat, edge_index, selections):
    mesh = plsc.VectorSubcoreMesh(
        core_axis_name="c", subcore_axis_name="s", num_cores=NC, num_subcores=NS
    )
    return pl.kernel(
        _edge_kernel,
        out_type=jax.ShapeDtypeStruct((NW, D_OUT, N_PAD), jnp.float32),
        mesh=mesh,
        compiler_params=pltpu.CompilerParams(
            use_tc_tiling_on_sc=False, needs_layout_passes=False
        ),
        scratch_types=[
            pltpu.VMEM((TAB_LEN,), jnp.int32),
            pltpu.VMEM((N_PAD,), jnp.float32),
            pltpu.VMEM((N_PAD,), jnp.float32),
            pltpu.VMEM((CHUNK,), jnp.int32),
            pltpu.VMEM((CHUNK,), jnp.int32),
            pltpu.VMEM((CHUNK,), jnp.int32),
            pltpu.VMEM((CHUNK,), jnp.int32),
            pltpu.VMEM((CHUNK,), jnp.int32),
            pltpu.VMEM((CHUNK,), jnp.int32),
            pltpu.SemaphoreType.DMA,
            pltpu.SemaphoreType.DMA,
            pltpu.SemaphoreType.DMA,
        ],
    )(table_flat, edge_index, selections)


def _bn_kernel(p_ref, b_ref, g_ref, be_ref, o_ref):
    p = p_ref[...].reshape(NW, D_OUT, NROW, 128)
    ob = jnp.sum(p, axis=0) + b_ref[...].reshape(D_OUT, 1, 1)
    ir = lax.broadcasted_iota(jnp.int32, (D_OUT, NROW, 128), 1)
    ic = lax.broadcasted_iota(jnp.int32, (D_OUT, NROW, 128), 2)
    msk = ir * 128 + ic < N_NODES
    inv_n = 1.0 / N_NODES
    mean = jnp.sum(jnp.where(msk, ob, 0.0), axis=(1, 2), keepdims=True) * inv_n
    ctr = jnp.where(msk, ob - mean, 0.0)
    var = jnp.sum(ctr * ctr, axis=(1, 2), keepdims=True) * inv_n
    xn = (ob - mean) * lax.rsqrt(var + BN_EPS)
    y = g_ref[...].reshape(D_OUT, 1, 1) * xn + be_ref[...].reshape(D_OUT, 1, 1)
    o_ref[...] = 0.3 * jax.nn.sigmoid(y)


def _bn_sigmoid(parts, b, gamma, beta):
    p2 = parts.reshape(NW * D_OUT * NROW, 128)
    return pl.pallas_call(
        _bn_kernel,
        out_shape=jax.ShapeDtypeStruct((D_OUT, NROW, 128), jnp.float32),
    )(p2, b.reshape(D_OUT, 1), gamma.reshape(D_OUT, 1), beta.reshape(D_OUT, 1))


@jax.jit
def kernel(x, edge_index, selections, W, b, gamma, beta):
    table = _build_table(x, W).reshape(TAB_LEN)
    parts = _scatter_edges(table, edge_index, selections)
    out = _bn_sigmoid(parts, b, gamma, beta)
    return out.reshape(D_OUT, N_PAD)[:, :N_NODES].T


# speedup vs baseline: 1.0113x; 1.0113x over previous
import jax
import jax.numpy as jnp
from jax import lax
from jax.experimental import pallas as pl
from jax.experimental.pallas import tpu as pltpu
from jax.experimental.pallas import tpu_sc as plsc

N_NODES = 10000
N_EDGES = 320000
D_IN = 128
D_OUT = 2
N_SEL = 9
BN_EPS = 1e-5

NC = 2
NS = 16
LANES = 16
NW = NC * NS
E_PER_W = N_EDGES // NW
CHUNK = 2000
N_CHUNK = E_PER_W // CHUNK
UNROLL = 8
N_PAD = 10240
NROW = N_PAD // 128
TAB_LEN = N_SEL * N_PAD
NB = 2048
GRID1 = N_PAD // NB


def _pack_kernel(x_ref, w_ref, o_ref):
    dn = (((1,), (1,)), ((), ()))
    a = lax.dot_general(w_ref[...], x_ref[...], dn,
                        preferred_element_type=jnp.float32)
    u0 = lax.bitcast_convert_type(a[:N_SEL].astype(jnp.bfloat16),
                                  jnp.uint16).astype(jnp.uint32)
    u1 = lax.bitcast_convert_type(a[N_SEL:].astype(jnp.bfloat16),
                                  jnp.uint16).astype(jnp.uint32)
    packed = lax.bitcast_convert_type((u1 << 16) | u0, jnp.int32)
    o_ref[...] = packed.reshape(N_SEL, NB // 128, 128)


def _build_table(x, W):
    w01 = W.transpose(1, 0, 2).reshape(D_OUT * N_SEL, D_IN)
    return pl.pallas_call(
        _pack_kernel,
        grid=(GRID1,),
        in_specs=[
            pl.BlockSpec((NB, D_IN), lambda i: (i, 0)),
            pl.BlockSpec((D_OUT * N_SEL, D_IN), lambda i: (0, 0)),
        ],
        out_specs=pl.BlockSpec((N_SEL, NB // 128, 128), lambda i: (0, i, 0)),
        out_shape=jax.ShapeDtypeStruct((N_SEL, NROW, 128), jnp.int32),
    )(x, w01)


def _edge_kernel(table_hbm, ei_hbm, sel_hbm, part_hbm,
                 table_v, acc0, acc1,
                 src_a, dst_a, sel_a, src_c, dst_c, sel_c,
                 sem_a, sem_c, sem_out):
    wid = lax.axis_index("s") * NC + lax.axis_index("c")
    bufs = ((src_a, dst_a, sel_a), (src_c, dst_c, sel_c))
    sems = (sem_a, sem_c)

    def issue(c, k):
        base = wid * E_PER_W + c * CHUNK
        return (
            pltpu.async_copy(ei_hbm.at[0, pl.ds(base, CHUNK)], bufs[k][0], sems[k]),
            pltpu.async_copy(ei_hbm.at[1, pl.ds(base, CHUNK)], bufs[k][1], sems[k]),
            pltpu.async_copy(sel_hbm.at[pl.ds(base, CHUNK)], bufs[k][2], sems[k]),
        )

    pend = issue(0, 0)
    tab_h = pltpu.async_copy(table_hbm, table_v, sem_out)

    zeros = jnp.zeros((LANES,), jnp.float32)

    @plsc.parallel_loop(0, N_PAD // LANES, step=1, unroll=8)
    def zero_body(i):
        acc0[pl.ds(i * LANES, LANES)] = zeros
        acc1[pl.ds(i * LANES, LANES)] = zeros

    tab_h.wait()

    for c in range(N_CHUNK):
        k = c & 1
        for h in pend:
            h.wait()
        if c + 1 < N_CHUNK:
            pend = issue(c + 1, 1 - k)
        src_b, dst_b, sel_b = bufs[k]

        @plsc.parallel_loop(0, CHUNK // LANES, step=1, unroll=UNROLL)
        def body(i):
            o = i * LANES
            s = src_b[pl.ds(o, LANES)]
            d = dst_b[pl.ds(o, LANES)]
            q = sel_b[pl.ds(o, LANES)]
            g = q * N_PAD + s
            w = plsc.load_gather(table_v, [g])
            c0 = plsc.bitcast(w << 16, jnp.float32)
            c1 = plsc.bitcast(w & jnp.int32(-65536), jnp.float32)
            plsc.addupdate_scatter(acc0, [d], c0)
            plsc.addupdate_scatter(acc1, [d], c1)

    o1 = pltpu.async_copy(acc0, part_hbm.at[wid, 0], sem_out)
    o2 = pltpu.async_copy(acc1, part_hbm.at[wid, 1], sem_out)
    o1.wait()
    o2.wait()


def _scatter_edges(table_flat, edge_index, selections):
    mesh = plsc.VectorSubcoreMesh(
        core_axis_name="c", subcore_axis_name="s", num_cores=NC, num_subcores=NS
    )
    return pl.kernel(
        _edge_kernel,
        out_type=jax.ShapeDtypeStruct((NW, D_OUT, N_PAD), jnp.float32),
        mesh=mesh,
        compiler_params=pltpu.CompilerParams(
            use_tc_tiling_on_sc=False, needs_layout_passes=False
        ),
        scratch_types=[
            pltpu.VMEM((TAB_LEN,), jnp.int32),
            pltpu.VMEM((N_PAD,), jnp.float32),
            pltpu.VMEM((N_PAD,), jnp.float32),
            pltpu.VMEM((CHUNK,), jnp.int32),
            pltpu.VMEM((CHUNK,), jnp.int32),
            pltpu.VMEM((CHUNK,), jnp.int32),
            pltpu.VMEM((CHUNK,), jnp.int32),
            pltpu.VMEM((CHUNK,), jnp.int32),
            pltpu.VMEM((CHUNK,), jnp.int32),
            pltpu.SemaphoreType.DMA,
            pltpu.SemaphoreType.DMA,
            pltpu.SemaphoreType.DMA,
        ],
    )(table_flat, edge_index, selections)


def _bn_kernel(p_ref, b_ref, g_ref, be_ref, o_ref):
    p = p_ref[...].reshape(NW, D_OUT, NROW, 128)
    ob = jnp.sum(p, axis=0) + b_ref[...].reshape(D_OUT, 1, 1)
    ir = lax.broadcasted_iota(jnp.int32, (D_OUT, NROW, 128), 1)
    ic = lax.broadcasted_iota(jnp.int32, (D_OUT, NROW, 128), 2)
    msk = ir * 128 + ic < N_NODES
    inv_n = 1.0 / N_NODES
    mean = jnp.sum(jnp.where(msk, ob, 0.0), axis=(1, 2), keepdims=True) * inv_n
    ctr = jnp.where(msk, ob - mean, 0.0)
    var = jnp.sum(ctr * ctr, axis=(1, 2), keepdims=True) * inv_n
    xn = (ob - mean) * lax.rsqrt(var + BN_EPS)
    y = g_ref[...].reshape(D_OUT, 1, 1) * xn + be_ref[...].reshape(D_OUT, 1, 1)
    o_ref[...] = 0.3 * jax.nn.sigmoid(y)


def _bn_sigmoid(parts, b, gamma, beta):
    p2 = parts.reshape(NW * D_OUT * NROW, 128)
    return pl.pallas_call(
        _bn_kernel,
        out_shape=jax.ShapeDtypeStruct((D_OUT, NROW, 128), jnp.float32),
    )(p2, b.reshape(D_OUT, 1), gamma.reshape(D_OUT, 1), beta.reshape(D_OUT, 1))


@jax.jit
def kernel(x, edge_index, selections, W, b, gamma, beta):
    table = _build_table(x, W).reshape(TAB_LEN)
    parts = _scatter_edges(table, edge_index, selections)
    out = _bn_sigmoid(parts, b, gamma, beta)
    return out.reshape(D_OUT, N_PAD)[:, :N_NODES].T


# speedup vs baseline: 1.0214x; 1.0100x over previous
import jax
import jax.numpy as jnp
from jax import lax
from jax.experimental import pallas as pl
from jax.experimental.pallas import tpu as pltpu
from jax.experimental.pallas import tpu_sc as plsc

N_NODES = 10000
N_EDGES = 320000
D_IN = 128
D_OUT = 2
N_SEL = 9
BN_EPS = 1e-5

NC = 2
NS = 16
LANES = 16
NW = NC * NS
E_PER_W = N_EDGES // NW
CHUNK = 2000
N_CHUNK = E_PER_W // CHUNK
UNROLL = 4
N_PAD = 10240
NROW = N_PAD // 128
TAB_LEN = N_SEL * N_PAD
NB = 2048
GRID1 = N_PAD // NB


def _pack_kernel(x_ref, w_ref, o_ref):
    dn = (((1,), (1,)), ((), ()))
    a = lax.dot_general(w_ref[...], x_ref[...], dn,
                        preferred_element_type=jnp.float32)
    u0 = lax.bitcast_convert_type(a[:N_SEL].astype(jnp.bfloat16),
                                  jnp.uint16).astype(jnp.uint32)
    u1 = lax.bitcast_convert_type(a[N_SEL:].astype(jnp.bfloat16),
                                  jnp.uint16).astype(jnp.uint32)
    packed = lax.bitcast_convert_type((u1 << 16) | u0, jnp.int32)
    o_ref[...] = packed.reshape(N_SEL, NB // 128, 128)


def _build_table(x, W):
    w01 = W.transpose(1, 0, 2).reshape(D_OUT * N_SEL, D_IN)
    return pl.pallas_call(
        _pack_kernel,
        grid=(GRID1,),
        in_specs=[
            pl.BlockSpec((NB, D_IN), lambda i: (i, 0)),
            pl.BlockSpec((D_OUT * N_SEL, D_IN), lambda i: (0, 0)),
        ],
        out_specs=pl.BlockSpec((N_SEL, NB // 128, 128), lambda i: (0, i, 0)),
        out_shape=jax.ShapeDtypeStruct((N_SEL, NROW, 128), jnp.int32),
    )(x, w01)


def _edge_kernel(table_hbm, ei_hbm, sel_hbm, part_hbm,
                 table_v, acc0, acc1,
                 src_a, dst_a, sel_a, src_c, dst_c, sel_c,
                 sem_a, sem_c, sem_out):
    wid = lax.axis_index("s") * NC + lax.axis_index("c")
    bufs = ((src_a, dst_a, sel_a), (src_c, dst_c, sel_c))
    sems = (sem_a, sem_c)

    def issue(c, k):
        base = wid * E_PER_W + c * CHUNK
        return (
            pltpu.async_copy(ei_hbm.at[0, pl.ds(base, CHUNK)], bufs[k][0], sems[k]),
            pltpu.async_copy(ei_hbm.at[1, pl.ds(base, CHUNK)], bufs[k][1], sems[k]),
            pltpu.async_copy(sel_hbm.at[pl.ds(base, CHUNK)], bufs[k][2], sems[k]),
        )

    pend = issue(0, 0)
    tab_h = pltpu.async_copy(table_hbm, table_v, sem_out)

    zeros = jnp.zeros((LANES,), jnp.float32)

    @plsc.parallel_loop(0, N_PAD // LANES, step=1, unroll=4)
    def zero_body(i):
        acc0[pl.ds(i * LANES, LANES)] = zeros
        acc1[pl.ds(i * LANES, LANES)] = zeros

    tab_h.wait()

    for c in range(N_CHUNK):
        k = c & 1
        for h in pend:
            h.wait()
        if c + 1 < N_CHUNK:
            pend = issue(c + 1, 1 - k)
        src_b, dst_b, sel_b = bufs[k]

        @plsc.parallel_loop(0, CHUNK // LANES, step=1, unroll=UNROLL)
        def body(i):
            o = i * LANES
            s = src_b[pl.ds(o, LANES)]
            d = dst_b[pl.ds(o, LANES)]
            q = sel_b[pl.ds(o, LANES)]
            g = q * N_PAD + s
            w = plsc.load_gather(table_v, [g])
            c0 = plsc.bitcast(w << 16, jnp.float32)
            c1 = plsc.bitcast(w & jnp.int32(-65536), jnp.float32)
            plsc.addupdate_scatter(acc0, [d], c0)
            plsc.addupdate_scatter(acc1, [d], c1)

    o1 = pltpu.async_copy(acc0, part_hbm.at[wid, 0], sem_out)
    o2 = pltpu.async_copy(acc1, part_hbm.at[wid, 1], sem_out)
    o1.wait()
    o2.wait()


def _scatter_edges(table_flat, edge_index, selections):
    mesh = plsc.VectorSubcoreMesh(
        core_axis_name="c", subcore_axis_name="s", num_cores=NC, num_subcores=NS
    )
    return pl.kernel(
        _edge_kernel,
        out_type=jax.ShapeDtypeStruct((NW, D_OUT, N_PAD), jnp.float32),
        mesh=mesh,
        compiler_params=pltpu.CompilerParams(
            use_tc_tiling_on_sc=False, needs_layout_passes=False
        ),
        scratch_types=[
            pltpu.VMEM((TAB_LEN,), jnp.int32),
            pltpu.VMEM((N_PAD,), jnp.float32),
            pltpu.VMEM((N_PAD,), jnp.float32),
            pltpu.VMEM((CHUNK,), jnp.int32),
            pltpu.VMEM((CHUNK,), jnp.int32),
            pltpu.VMEM((CHUNK,), jnp.int32),
            pltpu.VMEM((CHUNK,), jnp.int32),
            pltpu.VMEM((CHUNK,), jnp.int32),
            pltpu.VMEM((CHUNK,), jnp.int32),
            pltpu.SemaphoreType.DMA,
            pltpu.SemaphoreType.DMA,
            pltpu.SemaphoreType.DMA,
        ],
    )(table_flat, edge_index, selections)


def _bn_kernel(p_ref, b_ref, g_ref, be_ref, o_ref):
    p = p_ref[...].reshape(NW, D_OUT, NROW, 128)
    ob = jnp.sum(p, axis=0) + b_ref[...].reshape(D_OUT, 1, 1)
    ir = lax.broadcasted_iota(jnp.int32, (D_OUT, NROW, 128), 1)
    ic = lax.broadcasted_iota(jnp.int32, (D_OUT, NROW, 128), 2)
    msk = ir * 128 + ic < N_NODES
    inv_n = 1.0 / N_NODES
    mean = jnp.sum(jnp.where(msk, ob, 0.0), axis=(1, 2), keepdims=True) * inv_n
    ctr = jnp.where(msk, ob - mean, 0.0)
    var = jnp.sum(ctr * ctr, axis=(1, 2), keepdims=True) * inv_n
    xn = (ob - mean) * lax.rsqrt(var + BN_EPS)
    y = g_ref[...].reshape(D_OUT, 1, 1) * xn + be_ref[...].reshape(D_OUT, 1, 1)
    o_ref[...] = 0.3 * jax.nn.sigmoid(y)


def _bn_sigmoid(parts, b, gamma, beta):
    p2 = parts.reshape(NW * D_OUT * NROW, 128)
    return pl.pallas_call(
        _bn_kernel,
        out_shape=jax.ShapeDtypeStruct((D_OUT, NROW, 128), jnp.float32),
    )(p2, b.reshape(D_OUT, 1), gamma.reshape(D_OUT, 1), beta.reshape(D_OUT, 1))


@jax.jit
def kernel(x, edge_index, selections, W, b, gamma, beta):
    table = _build_table(x, W).reshape(TAB_LEN)
    parts = _scatter_edges(table, edge_index, selections)
    out = _bn_sigmoid(parts, b, gamma, beta)
    return out.reshape(D_OUT, N_PAD)[:, :N_NODES].T


# speedup vs baseline: 1.0629x; 1.0405x over previous
import jax
import jax.numpy as jnp
from jax import lax
from jax.experimental import pallas as pl
from jax.experimental.pallas import tpu as pltpu
from jax.experimental.pallas import tpu_sc as plsc

N_NODES = 10000
N_EDGES = 320000
D_IN = 128
D_OUT = 2
N_SEL = 9
BN_EPS = 1e-5

NC = 2
NS = 16
LANES = 16
NW = NC * NS
E_PER_W = N_EDGES // NW
CHUNK = 2000
N_CHUNK = E_PER_W // CHUNK
UNROLL = 4
N_PAD = 10240
NROW = N_PAD // 128
TAB_LEN = N_SEL * N_PAD
NB = 5120
GRID1 = N_PAD // NB


def _pack_kernel(x_ref, w_ref, o_ref):
    dn = (((1,), (1,)), ((), ()))
    a = lax.dot_general(w_ref[...], x_ref[...], dn,
                        preferred_element_type=jnp.float32)
    u0 = lax.bitcast_convert_type(a[:N_SEL].astype(jnp.bfloat16),
                                  jnp.uint16).astype(jnp.uint32)
    u1 = lax.bitcast_convert_type(a[N_SEL:].astype(jnp.bfloat16),
                                  jnp.uint16).astype(jnp.uint32)
    packed = lax.bitcast_convert_type((u1 << 16) | u0, jnp.int32)
    o_ref[...] = packed.reshape(N_SEL, NB // 128, 128)


def _build_table(x, W):
    w01 = W.transpose(1, 0, 2).reshape(D_OUT * N_SEL, D_IN)
    return pl.pallas_call(
        _pack_kernel,
        grid=(GRID1,),
        in_specs=[
            pl.BlockSpec((NB, D_IN), lambda i: (i, 0)),
            pl.BlockSpec((D_OUT * N_SEL, D_IN), lambda i: (0, 0)),
        ],
        out_specs=pl.BlockSpec((N_SEL, NB // 128, 128), lambda i: (0, i, 0)),
        out_shape=jax.ShapeDtypeStruct((N_SEL, NROW, 128), jnp.int32),
    )(x, w01)


def _edge_kernel(table_hbm, ei_hbm, sel_hbm, part_hbm,
                 table_v, acc0, acc1,
                 src_a, dst_a, sel_a, src_c, dst_c, sel_c,
                 sem_a, sem_c, sem_out):
    wid = lax.axis_index("s") * NC + lax.axis_index("c")
    bufs = ((src_a, dst_a, sel_a), (src_c, dst_c, sel_c))
    sems = (sem_a, sem_c)

    def issue(c, k):
        base = wid * E_PER_W + c * CHUNK
        return (
            pltpu.async_copy(ei_hbm.at[0, pl.ds(base, CHUNK)], bufs[k][0], sems[k]),
            pltpu.async_copy(ei_hbm.at[1, pl.ds(base, CHUNK)], bufs[k][1], sems[k]),
            pltpu.async_copy(sel_hbm.at[pl.ds(base, CHUNK)], bufs[k][2], sems[k]),
        )

    pend = issue(0, 0)
    tab_h = pltpu.async_copy(table_hbm, table_v, sem_out)

    zeros = jnp.zeros((LANES,), jnp.float32)

    @plsc.parallel_loop(0, N_PAD // LANES, step=1, unroll=4)
    def zero_body(i):
        acc0[pl.ds(i * LANES, LANES)] = zeros
        acc1[pl.ds(i * LANES, LANES)] = zeros

    tab_h.wait()

    for c in range(N_CHUNK):
        k = c & 1
        for h in pend:
            h.wait()
        if c + 1 < N_CHUNK:
            pend = issue(c + 1, 1 - k)
        src_b, dst_b, sel_b = bufs[k]

        @plsc.parallel_loop(0, CHUNK // LANES, step=1, unroll=UNROLL)
        def body(i):
            o = i * LANES
            s = src_b[pl.ds(o, LANES)]
            d = dst_b[pl.ds(o, LANES)]
            q = sel_b[pl.ds(o, LANES)]
            g = q * N_PAD + s
            w = plsc.load_gather(table_v, [g])
            c0 = plsc.bitcast(w << 16, jnp.float32)
            c1 = plsc.bitcast(w & jnp.int32(-65536), jnp.float32)
            plsc.addupdate_scatter(acc0, [d], c0)
            plsc.addupdate_scatter(acc1, [d], c1)

    o1 = pltpu.async_copy(acc0, part_hbm.at[wid, 0], sem_out)
    o2 = pltpu.async_copy(acc1, part_hbm.at[wid, 1], sem_out)
    o1.wait()
    o2.wait()


def _scatter_edges(table_flat, edge_index, selections):
    mesh = plsc.VectorSubcoreMesh(
        core_axis_name="c", subcore_axis_name="s", num_cores=NC, num_subcores=NS
    )
    return pl.kernel(
        _edge_kernel,
        out_type=jax.ShapeDtypeStruct((NW, D_OUT, N_PAD), jnp.float32),
        mesh=mesh,
        compiler_params=pltpu.CompilerParams(
            use_tc_tiling_on_sc=False, needs_layout_passes=False
        ),
        scratch_types=[
            pltpu.VMEM((TAB_LEN,), jnp.int32),
            pltpu.VMEM((N_PAD,), jnp.float32),
            pltpu.VMEM((N_PAD,), jnp.float32),
            pltpu.VMEM((CHUNK,), jnp.int32),
            pltpu.VMEM((CHUNK,), jnp.int32),
            pltpu.VMEM((CHUNK,), jnp.int32),
            pltpu.VMEM((CHUNK,), jnp.int32),
            pltpu.VMEM((CHUNK,), jnp.int32),
            pltpu.VMEM((CHUNK,), jnp.int32),
            pltpu.SemaphoreType.DMA,
            pltpu.SemaphoreType.DMA,
            pltpu.SemaphoreType.DMA,
        ],
    )(table_flat, edge_index, selections)


def _bn_kernel(p_ref, b_ref, g_ref, be_ref, o_ref):
    p = p_ref[...].reshape(NW, D_OUT, NROW, 128)
    ob = jnp.sum(p, axis=0) + b_ref[...].reshape(D_OUT, 1, 1)
    ir = lax.broadcasted_iota(jnp.int32, (D_OUT, NROW, 128), 1)
    ic = lax.broadcasted_iota(jnp.int32, (D_OUT, NROW, 128), 2)
    msk = ir * 128 + ic < N_NODES
    inv_n = 1.0 / N_NODES
    mean = jnp.sum(jnp.where(msk, ob, 0.0), axis=(1, 2), keepdims=True) * inv_n
    ctr = jnp.where(msk, ob - mean, 0.0)
    var = jnp.sum(ctr * ctr, axis=(1, 2), keepdims=True) * inv_n
    xn = (ob - mean) * lax.rsqrt(var + BN_EPS)
    y = g_ref[...].reshape(D_OUT, 1, 1) * xn + be_ref[...].reshape(D_OUT, 1, 1)
    o_ref[...] = 0.3 * jax.nn.sigmoid(y)


def _bn_sigmoid(parts, b, gamma, beta):
    p2 = parts.reshape(NW * D_OUT * NROW, 128)
    return pl.pallas_call(
        _bn_kernel,
        out_shape=jax.ShapeDtypeStruct((D_OUT, NROW, 128), jnp.float32),
    )(p2, b.reshape(D_OUT, 1), gamma.reshape(D_OUT, 1), beta.reshape(D_OUT, 1))


@jax.jit
def kernel(x, edge_index, selections, W, b, gamma, beta):
    table = _build_table(x, W).reshape(TAB_LEN)
    parts = _scatter_edges(table, edge_index, selections)
    out = _bn_sigmoid(parts, b, gamma, beta)
    return out.reshape(D_OUT, N_PAD)[:, :N_NODES].T
